# Initial kernel scaffold; baseline (speedup 1.0000x reference)
import functools

import jax
import jax.numpy as jnp
from jax import lax
from jax.experimental import pallas as pl
from jax.experimental.pallas import tpu as pltpu
from jax.experimental.pallas import tpu_sc as plsc

NC = 2
NS = 16
NW = NC * NS
CH = 128

K = 16
D = 128
WD = 32


def _wid():
    return lax.axis_index("s") * NC + lax.axis_index("c")


def _sc_mesh():
    return plsc.VectorSubcoreMesh(core_axis_name="c", subcore_axis_name="s")


def _gather_level1(ids, neigh_ids, path_types, node_features):
    B = ids.shape[0]
    bpw = B // NW
    assert B % NW == 0 and bpw <= CH and bpw % 8 == 0

    @functools.partial(
        pl.kernel,
        out_type=(
            jax.ShapeDtypeStruct((B, K), jnp.int32),
            jax.ShapeDtypeStruct((B, K), jnp.int32),
            jax.ShapeDtypeStruct((B, D), jnp.float32),
        ),
        mesh=_sc_mesh(),
        scratch_types=[
            pltpu.VMEM((bpw,), jnp.int32),
            pltpu.VMEM((bpw, K), jnp.int32),
            pltpu.VMEM((bpw, K), jnp.int32),
            pltpu.VMEM((bpw, D), jnp.float32),
            pltpu.SemaphoreType.DMA,
            pltpu.SemaphoreType.DMA,
            pltpu.SemaphoreType.DMA,
        ],
    )
    def k(ids_h, nb_h, pt_h, ft_h, n1_o, pt0_o, x0_o,
          idx_v, n1_v, pt_v, x_v, s1, s2, s3):
        base = _wid() * bpw
        pltpu.sync_copy(ids_h.at[pl.ds(base, bpw)], idx_v)
        c1 = pltpu.async_copy(nb_h.at[idx_v], n1_v, s1)
        c2 = pltpu.async_copy(pt_h.at[idx_v], pt_v, s2)
        c3 = pltpu.async_copy(ft_h.at[idx_v], x_v, s3)
        c1.wait()
        c2.wait()
        c3.wait()
        pltpu.sync_copy(n1_v, n1_o.at[pl.ds(base, bpw)])
        pltpu.sync_copy(pt_v, pt0_o.at[pl.ds(base, bpw)])
        pltpu.sync_copy(x_v, x0_o.at[pl.ds(base, bpw)])

    return k(ids, neigh_ids, path_types, node_features)


def _gather_level2(idx2, neigh_ids, path_types):
    B = idx2.shape[0] * CH
    bpw = B // NW
    nch = bpw // CH
    assert bpw % CH == 0

    @functools.partial(
        pl.kernel,
        out_type=(
            jax.ShapeDtypeStruct((B, K), jnp.int32),
            jax.ShapeDtypeStruct((B, K), jnp.int32),
        ),
        mesh=_sc_mesh(),
        scratch_types=[
            pltpu.VMEM((nch, CH), jnp.int32),
            pltpu.VMEM((bpw, K), jnp.int32),
            pltpu.VMEM((bpw, K), jnp.int32),
            pltpu.SemaphoreType.DMA,
            pltpu.SemaphoreType.DMA,
        ],
    )
    def k(idx_h, nb_h, pt_h, n2_o, pt1_o, idx_v, n2_v, pt_v, s1, s2):
        base = _wid() * bpw
        pltpu.sync_copy(idx_h.at[pl.ds(base // CH, nch)], idx_v)
        for c in range(nch):
            lo = c * CH
            iv = idx_v.at[c]
            c1 = pltpu.async_copy(nb_h.at[iv], n2_v.at[pl.ds(lo, CH)], s1)
            c2 = pltpu.async_copy(pt_h.at[iv], pt_v.at[pl.ds(lo, CH)], s2)
            c1.wait()
            c2.wait()
        pltpu.sync_copy(n2_v, n2_o.at[pl.ds(base, bpw)])
        pltpu.sync_copy(pt_v, pt1_o.at[pl.ds(base, bpw)])

    return k(idx2, neigh_ids, path_types)


def _gather_features(idx2, node_features):
    B = idx2.shape[0] * CH
    bpw = B // NW
    nch = bpw // CH
    assert bpw % CH == 0 and nch % 2 == 0

    @functools.partial(
        pl.kernel,
        out_type=jax.ShapeDtypeStruct((B, D), jnp.float32),
        mesh=_sc_mesh(),
        scratch_types=[
            pltpu.VMEM((nch, CH), jnp.int32),
            pltpu.VMEM((CH, D), jnp.float32),
            pltpu.VMEM((CH, D), jnp.float32),
            pltpu.SemaphoreType.DMA,
            pltpu.SemaphoreType.DMA,
        ],
    )
    def k(idx_h, ft_h, x_o, idx_v, buf0, buf1, s0, s1):
        base = _wid() * bpw
        pltpu.sync_copy(idx_h.at[pl.ds(base // CH, nch)], idx_v)
        pltpu.async_copy(ft_h.at[idx_v.at[0]], buf0, s0)

        def body(p, _):
            c0 = 2 * p
            pltpu.async_copy(ft_h.at[idx_v.at[c0 + 1]], buf1, s1)
            pltpu.make_async_copy(ft_h.at[pl.ds(0, CH)], buf0, s0).wait()
            pltpu.sync_copy(buf0, x_o.at[pl.ds(base + c0 * CH, CH)])

            @pl.when(c0 + 2 < nch)
            def _():
                pltpu.async_copy(ft_h.at[idx_v.at[c0 + 2]], buf0, s0)

            pltpu.make_async_copy(ft_h.at[pl.ds(0, CH)], buf1, s1).wait()
            pltpu.sync_copy(buf1, x_o.at[pl.ds(base + (c0 + 1) * CH, CH)])
            return 0

        lax.fori_loop(0, nch // 2, body, 0)

    return k(idx2, node_features)


def _amp_table(wep_ref, wp_ref, bp_ref):
    pre = jnp.dot(wep_ref[:], wp_ref[:], preferred_element_type=jnp.float32)
    return jax.nn.sigmoid(pre + bp_ref[:])


def _weighted_mean(pt, amp_table, nv_ref, rb):
    acc = None
    for k in range(K):
        oh = (pt[:, k][:, None]
              == lax.broadcasted_iota(jnp.int32, (rb, D), 1)).astype(jnp.float32)
        ampk = jnp.dot(oh, amp_table, preferred_element_type=jnp.float32)
        term = ampk * nv_ref[:, k, :]
        acc = term if acc is None else acc + term
    return acc * (1.0 / K)


def _hop1_layer1(x2r, pt1, x1, wep, wp1, bp1, wn1, ws1, ba1):
    B2 = x1.shape[0]
    RB = 512
    grid = B2 // RB

    def body(x2_ref, pt_ref, x1_ref, wep_ref, wp1_ref, bp1_ref,
             wn1_ref, ws1_ref, ba1_ref, h1_ref):
        amp = _amp_table(wep_ref, wp1_ref, bp1_ref)
        nm = _weighted_mean(pt_ref[:], amp, x2_ref, RB)
        h = (jnp.dot(nm, wn1_ref[:], preferred_element_type=jnp.float32)
             + jnp.dot(x1_ref[:], ws1_ref[:], preferred_element_type=jnp.float32)
             + ba1_ref[:])
        h1_ref[:] = jnp.maximum(h, 0.0)

    return pl.pallas_call(
        body,
        grid=(grid,),
        in_specs=[
            pl.BlockSpec((RB, K, D), lambda i: (i, 0, 0)),
            pl.BlockSpec((RB, K), lambda i: (i, 0)),
            pl.BlockSpec((RB, D), lambda i: (i, 0)),
            pl.BlockSpec((D, WD), lambda i: (0, 0)),
            pl.BlockSpec((WD, D), lambda i: (0, 0)),
            pl.BlockSpec((1, D), lambda i: (0, 0)),
            pl.BlockSpec((D, D), lambda i: (0, 0)),
            pl.BlockSpec((D, D), lambda i: (0, 0)),
            pl.BlockSpec((1, D), lambda i: (0, 0)),
        ],
        out_specs=pl.BlockSpec((RB, D), lambda i: (i, 0)),
        out_shape=jax.ShapeDtypeStruct((B2, D), jnp.float32),
    )(x2r, pt1, x1, wep, wp1, bp1, wn1, ws1, ba1)


def _final_layer(x0, pt0, x1r, h1r, wep, wp1, bp1, wn1, ws1, ba1,
                 wp2, bp2, wn2, ws2, ba2):
    B = x0.shape[0]
    RB = 512
    grid = B // RB

    def body(x0_ref, pt_ref, x1_ref, h1_ref, wep_ref, wp1_ref, bp1_ref,
             wn1_ref, ws1_ref, ba1_ref, wp2_ref, bp2_ref, wn2_ref,
             ws2_ref, ba2_ref, out_ref):
        pt = pt_ref[:]
        amp1 = _amp_table(wep_ref, wp1_ref, bp1_ref)
        nm0 = _weighted_mean(pt, amp1, x1_ref, RB)
        h0 = (jnp.dot(nm0, wn1_ref[:], preferred_element_type=jnp.float32)
              + jnp.dot(x0_ref[:], ws1_ref[:], preferred_element_type=jnp.float32)
              + ba1_ref[:])
        h0 = jnp.maximum(h0, 0.0)
        amp2 = _amp_table(wep_ref, wp2_ref, bp2_ref)
        nm1 = _weighted_mean(pt, amp2, h1_ref, RB)
        out = (jnp.dot(nm1, wn2_ref[:], preferred_element_type=jnp.float32)
               + jnp.dot(h0, ws2_ref[:], preferred_element_type=jnp.float32)
               + ba2_ref[:])
        nrm = jnp.sqrt(jnp.sum(out * out, axis=1, keepdims=True))
        out_ref[:] = out / jnp.maximum(nrm, 1e-12)

    return pl.pallas_call(
        body,
        grid=(grid,),
        in_specs=[
            pl.BlockSpec((RB, D), lambda i: (i, 0)),
            pl.BlockSpec((RB, K), lambda i: (i, 0)),
            pl.BlockSpec((RB, K, D), lambda i: (i, 0, 0)),
            pl.BlockSpec((RB, K, D), lambda i: (i, 0, 0)),
            pl.BlockSpec((D, WD), lambda i: (0, 0)),
            pl.BlockSpec((WD, D), lambda i: (0, 0)),
            pl.BlockSpec((1, D), lambda i: (0, 0)),
            pl.BlockSpec((D, D), lambda i: (0, 0)),
            pl.BlockSpec((D, D), lambda i: (0, 0)),
            pl.BlockSpec((1, D), lambda i: (0, 0)),
            pl.BlockSpec((WD, D), lambda i: (0, 0)),
            pl.BlockSpec((1, D), lambda i: (0, 0)),
            pl.BlockSpec((D, D), lambda i: (0, 0)),
            pl.BlockSpec((D, D), lambda i: (0, 0)),
            pl.BlockSpec((1, D), lambda i: (0, 0)),
        ],
        out_specs=pl.BlockSpec((RB, D), lambda i: (i, 0)),
        out_shape=jax.ShapeDtypeStruct((B, D), jnp.float32),
    )(x0, pt0, x1r, h1r, wep, wp1, bp1, wn1, ws1, ba1,
      wp2, bp2, wn2, ws2, ba2)


def kernel(batch_keys, batch_labels, batch_negs, path_types, neigh_ids,
           node_features, walk_embeddings, weight_self_1, weight_neigh_1,
           weight_path_1, bias_path_1, bias_aggregate_1, weight_self_2,
           weight_neigh_2, weight_path_2, bias_path_2, bias_aggregate_2):
    B0 = batch_keys.shape[0]
    ids = jnp.concatenate([batch_keys, batch_labels, batch_negs])
    B = ids.shape[0]

    n1, pt0, x0 = _gather_level1(ids, neigh_ids, path_types, node_features)
    n1f2 = n1.reshape(B * K // CH, CH)
    n2, pt1 = _gather_level2(n1f2, neigh_ids, path_types)
    x1 = _gather_features(n1f2, node_features)
    x2 = _gather_features(n2.reshape(B * K * K // CH, CH), node_features)

    wep = jnp.zeros((D, WD), jnp.float32).at[:walk_embeddings.shape[0]].set(
        walk_embeddings)
    bp1 = bias_path_1.reshape(1, D)
    ba1 = bias_aggregate_1.reshape(1, D)
    bp2 = bias_path_2.reshape(1, D)
    ba2 = bias_aggregate_2.reshape(1, D)

    h1 = _hop1_layer1(x2.reshape(B * K, K, D), pt1, x1, wep,
                      weight_path_1, bp1, weight_neigh_1, weight_self_1, ba1)
    out = _final_layer(x0, pt0, x1.reshape(B, K, D), h1.reshape(B, K, D),
                       wep, weight_path_1, bp1, weight_neigh_1,
                       weight_self_1, ba1, weight_path_2, bp2,
                       weight_neigh_2, weight_self_2, ba2)
    return (out[:B0], out[B0:2 * B0], out[2 * B0:])


# speedup vs baseline: 4.2661x; 4.2661x over previous
import functools

import jax
import jax.numpy as jnp
from jax import lax
from jax.experimental import pallas as pl
from jax.experimental.pallas import tpu as pltpu
from jax.experimental.pallas import tpu_sc as plsc

NC = 2
NS = 16
NW = NC * NS
CH = 128

K = 16
D = 128
WD = 32


def _wid():
    return lax.axis_index("s") * NC + lax.axis_index("c")


def _sc_mesh():
    return plsc.VectorSubcoreMesh(core_axis_name="c", subcore_axis_name="s")


def _gather_level1(ids, neigh_ids, path_types, node_features):
    B = ids.shape[0]
    bpw = B // NW
    assert B % NW == 0 and bpw <= CH and bpw % 8 == 0

    @functools.partial(
        pl.kernel,
        out_type=(
            jax.ShapeDtypeStruct((B, K), jnp.int32),
            jax.ShapeDtypeStruct((B, K), jnp.int32),
            jax.ShapeDtypeStruct((B, D), jnp.float32),
        ),
        mesh=_sc_mesh(),
        scratch_types=[
            pltpu.VMEM((bpw,), jnp.int32),
            pltpu.VMEM((bpw, K), jnp.int32),
            pltpu.VMEM((bpw, K), jnp.int32),
            pltpu.VMEM((bpw, D), jnp.float32),
            pltpu.SemaphoreType.DMA,
            pltpu.SemaphoreType.DMA,
            pltpu.SemaphoreType.DMA,
        ],
    )
    def k(ids_h, nb_h, pt_h, ft_h, n1_o, pt0_o, x0_o,
          idx_v, n1_v, pt_v, x_v, s1, s2, s3):
        base = _wid() * bpw
        pltpu.sync_copy(ids_h.at[pl.ds(base, bpw)], idx_v)
        c1 = pltpu.async_copy(nb_h.at[idx_v], n1_v, s1)
        c2 = pltpu.async_copy(pt_h.at[idx_v], pt_v, s2)
        c3 = pltpu.async_copy(ft_h.at[idx_v], x_v, s3)
        c1.wait()
        c2.wait()
        c3.wait()
        pltpu.sync_copy(n1_v, n1_o.at[pl.ds(base, bpw)])
        pltpu.sync_copy(pt_v, pt0_o.at[pl.ds(base, bpw)])
        pltpu.sync_copy(x_v, x0_o.at[pl.ds(base, bpw)])

    return k(ids, neigh_ids, path_types, node_features)


def _gather_level2(idx2, neigh_ids, path_types):
    B = idx2.shape[0] * CH
    bpw = B // NW
    nch = bpw // CH
    assert bpw % CH == 0

    @functools.partial(
        pl.kernel,
        out_type=(
            jax.ShapeDtypeStruct((B, K), jnp.int32),
            jax.ShapeDtypeStruct((B, K), jnp.int32),
        ),
        mesh=_sc_mesh(),
        scratch_types=[
            pltpu.VMEM((nch, CH), jnp.int32),
            pltpu.VMEM((bpw, K), jnp.int32),
            pltpu.VMEM((bpw, K), jnp.int32),
            pltpu.SemaphoreType.DMA,
            pltpu.SemaphoreType.DMA,
        ],
    )
    def k(idx_h, nb_h, pt_h, n2_o, pt1_o, idx_v, n2_v, pt_v, s1, s2):
        base = _wid() * bpw
        pltpu.sync_copy(idx_h.at[pl.ds(base // CH, nch)], idx_v)
        for c in range(nch):
            lo = c * CH
            iv = idx_v.at[c]
            c1 = pltpu.async_copy(nb_h.at[iv], n2_v.at[pl.ds(lo, CH)], s1)
            c2 = pltpu.async_copy(pt_h.at[iv], pt_v.at[pl.ds(lo, CH)], s2)
            c1.wait()
            c2.wait()
        pltpu.sync_copy(n2_v, n2_o.at[pl.ds(base, bpw)])
        pltpu.sync_copy(pt_v, pt1_o.at[pl.ds(base, bpw)])

    return k(idx2, neigh_ids, path_types)


def _gather_features(idx, node_features):
    B = idx.shape[0]
    bpw = B // NW
    nch = bpw // CH
    assert bpw % CH == 0 and (nch == 1 or nch % 2 == 0)

    @functools.partial(
        pl.kernel,
        out_type=jax.ShapeDtypeStruct((B, D), jnp.float32),
        mesh=_sc_mesh(),
        scratch_types=[
            pltpu.VMEM((bpw,), jnp.int32),
            pltpu.VMEM((CH, D), jnp.float32),
            pltpu.VMEM((CH, D), jnp.float32),
            pltpu.SemaphoreType.DMA,
            pltpu.SemaphoreType.DMA,
        ],
    )
    def k(idx_h, ft_h, x_o, idx_v, buf0, buf1, s0, s1):
        base = _wid() * bpw
        pltpu.sync_copy(idx_h.at[pl.ds(base, bpw)], idx_v)
        pltpu.async_copy(ft_h.at[idx_v.at[pl.ds(0, CH)]], buf0, s0)
        if nch == 1:
            pltpu.make_async_copy(ft_h.at[pl.ds(0, CH)], buf0, s0).wait()
            pltpu.sync_copy(buf0, x_o.at[pl.ds(base, CH)])
            return

        def body(p, _):
            c0 = 2 * p
            pltpu.async_copy(
                ft_h.at[idx_v.at[pl.ds((c0 + 1) * CH, CH)]], buf1, s1)
            pltpu.make_async_copy(ft_h.at[pl.ds(0, CH)], buf0, s0).wait()
            pltpu.sync_copy(buf0, x_o.at[pl.ds(base + c0 * CH, CH)])

            @pl.when(c0 + 2 < nch)
            def _():
                pltpu.async_copy(
                    ft_h.at[idx_v.at[pl.ds((c0 + 2) * CH, CH)]], buf0, s0)

            pltpu.make_async_copy(ft_h.at[pl.ds(0, CH)], buf1, s1).wait()
            pltpu.sync_copy(buf1, x_o.at[pl.ds(base + (c0 + 1) * CH, CH)])
            return 0

        lax.fori_loop(0, nch // 2, body, 0)

    return k(idx, node_features)


def _amp_table(wep_ref, wp_ref, bp_ref):
    pre = jnp.dot(wep_ref[:], wp_ref[:], preferred_element_type=jnp.float32)
    return jax.nn.sigmoid(pre + bp_ref[:])


def _weighted_mean(pt, amp_table, nv_ref, rb):
    acc = None
    for k in range(K):
        oh = (pt[:, k][:, None]
              == lax.broadcasted_iota(jnp.int32, (rb, D), 1)).astype(jnp.float32)
        ampk = jnp.dot(oh, amp_table, preferred_element_type=jnp.float32)
        term = ampk * nv_ref[:, k, :]
        acc = term if acc is None else acc + term
    return acc * (1.0 / K)


def _hop1_layer1(x2r, pt1, x1, wep, wp1, bp1, wn1, ws1, ba1):
    B2 = x1.shape[0]
    RB = 512
    grid = B2 // RB

    def body(x2_ref, pt_ref, x1_ref, wep_ref, wp1_ref, bp1_ref,
             wn1_ref, ws1_ref, ba1_ref, h1_ref):
        amp = _amp_table(wep_ref, wp1_ref, bp1_ref)
        nm = _weighted_mean(pt_ref[:], amp, x2_ref, RB)
        h = (jnp.dot(nm, wn1_ref[:], preferred_element_type=jnp.float32)
             + jnp.dot(x1_ref[:], ws1_ref[:], preferred_element_type=jnp.float32)
             + ba1_ref[:])
        h1_ref[:] = jnp.maximum(h, 0.0)

    return pl.pallas_call(
        body,
        grid=(grid,),
        in_specs=[
            pl.BlockSpec((RB, K, D), lambda i: (i, 0, 0)),
            pl.BlockSpec((RB, K), lambda i: (i, 0)),
            pl.BlockSpec((RB, D), lambda i: (i, 0)),
            pl.BlockSpec((D, WD), lambda i: (0, 0)),
            pl.BlockSpec((WD, D), lambda i: (0, 0)),
            pl.BlockSpec((1, D), lambda i: (0, 0)),
            pl.BlockSpec((D, D), lambda i: (0, 0)),
            pl.BlockSpec((D, D), lambda i: (0, 0)),
            pl.BlockSpec((1, D), lambda i: (0, 0)),
        ],
        out_specs=pl.BlockSpec((RB, D), lambda i: (i, 0)),
        out_shape=jax.ShapeDtypeStruct((B2, D), jnp.float32),
    )(x2r, pt1, x1, wep, wp1, bp1, wn1, ws1, ba1)


def _final_layer(x0, pt0, x1r, h1r, wep, wp1, bp1, wn1, ws1, ba1,
                 wp2, bp2, wn2, ws2, ba2):
    B = x0.shape[0]
    RB = 512
    grid = B // RB

    def body(x0_ref, pt_ref, x1_ref, h1_ref, wep_ref, wp1_ref, bp1_ref,
             wn1_ref, ws1_ref, ba1_ref, wp2_ref, bp2_ref, wn2_ref,
             ws2_ref, ba2_ref, out_ref):
        pt = pt_ref[:]
        amp1 = _amp_table(wep_ref, wp1_ref, bp1_ref)
        nm0 = _weighted_mean(pt, amp1, x1_ref, RB)
        h0 = (jnp.dot(nm0, wn1_ref[:], preferred_element_type=jnp.float32)
              + jnp.dot(x0_ref[:], ws1_ref[:], preferred_element_type=jnp.float32)
              + ba1_ref[:])
        h0 = jnp.maximum(h0, 0.0)
        amp2 = _amp_table(wep_ref, wp2_ref, bp2_ref)
        nm1 = _weighted_mean(pt, amp2, h1_ref, RB)
        out = (jnp.dot(nm1, wn2_ref[:], preferred_element_type=jnp.float32)
               + jnp.dot(h0, ws2_ref[:], preferred_element_type=jnp.float32)
               + ba2_ref[:])
        nrm = jnp.sqrt(jnp.sum(out * out, axis=1, keepdims=True))
        out_ref[:] = out / jnp.maximum(nrm, 1e-12)

    return pl.pallas_call(
        body,
        grid=(grid,),
        in_specs=[
            pl.BlockSpec((RB, D), lambda i: (i, 0)),
            pl.BlockSpec((RB, K), lambda i: (i, 0)),
            pl.BlockSpec((RB, K, D), lambda i: (i, 0, 0)),
            pl.BlockSpec((RB, K, D), lambda i: (i, 0, 0)),
            pl.BlockSpec((D, WD), lambda i: (0, 0)),
            pl.BlockSpec((WD, D), lambda i: (0, 0)),
            pl.BlockSpec((1, D), lambda i: (0, 0)),
            pl.BlockSpec((D, D), lambda i: (0, 0)),
            pl.BlockSpec((D, D), lambda i: (0, 0)),
            pl.BlockSpec((1, D), lambda i: (0, 0)),
            pl.BlockSpec((WD, D), lambda i: (0, 0)),
            pl.BlockSpec((1, D), lambda i: (0, 0)),
            pl.BlockSpec((D, D), lambda i: (0, 0)),
            pl.BlockSpec((D, D), lambda i: (0, 0)),
            pl.BlockSpec((1, D), lambda i: (0, 0)),
        ],
        out_specs=pl.BlockSpec((RB, D), lambda i: (i, 0)),
        out_shape=jax.ShapeDtypeStruct((B, D), jnp.float32),
    )(x0, pt0, x1r, h1r, wep, wp1, bp1, wn1, ws1, ba1,
      wp2, bp2, wn2, ws2, ba2)


def kernel(batch_keys, batch_labels, batch_negs, path_types, neigh_ids,
           node_features, walk_embeddings, weight_self_1, weight_neigh_1,
           weight_path_1, bias_path_1, bias_aggregate_1, weight_self_2,
           weight_neigh_2, weight_path_2, bias_path_2, bias_aggregate_2):
    B0 = batch_keys.shape[0]
    ids = jnp.concatenate([batch_keys, batch_labels, batch_negs])
    B = ids.shape[0]

    n1 = jnp.take(neigh_ids, ids, axis=0)
    pt0 = jnp.take(path_types, ids, axis=0)
    x0 = _gather_features(jnp.pad(ids, (0, CH * NW - B)), node_features)[:B]
    n1f = n1.reshape(B * K)
    n2 = jnp.take(neigh_ids, n1f, axis=0)
    pt1 = jnp.take(path_types, n1f, axis=0)
    x1 = _gather_features(n1f, node_features)
    x2 = _gather_features(n2.reshape(B * K * K), node_features)

    wep = jnp.zeros((D, WD), jnp.float32).at[:walk_embeddings.shape[0]].set(
        walk_embeddings)
    bp1 = bias_path_1.reshape(1, D)
    ba1 = bias_aggregate_1.reshape(1, D)
    bp2 = bias_path_2.reshape(1, D)
    ba2 = bias_aggregate_2.reshape(1, D)

    h1 = _hop1_layer1(x2.reshape(B * K, K, D), pt1, x1, wep,
                      weight_path_1, bp1, weight_neigh_1, weight_self_1, ba1)
    out = _final_layer(x0, pt0, x1.reshape(B, K, D), h1.reshape(B, K, D),
                       wep, weight_path_1, bp1, weight_neigh_1,
                       weight_self_1, ba1, weight_path_2, bp2,
                       weight_neigh_2, weight_self_2, ba2)
    return (out[:B0], out[B0:2 * B0], out[2 * B0:])


# speedup vs baseline: 5.3559x; 1.2555x over previous
import functools

import jax
import jax.numpy as jnp
from jax import lax
from jax.experimental import pallas as pl
from jax.experimental.pallas import tpu as pltpu
from jax.experimental.pallas import tpu_sc as plsc

NC = 2
NS = 16
NW = NC * NS
CH = 128

K = 16
D = 128
WD = 32


def _wid():
    return lax.axis_index("s") * NC + lax.axis_index("c")


def _sc_mesh():
    return plsc.VectorSubcoreMesh(core_axis_name="c", subcore_axis_name="s")


def _gather_level1(ids, neigh_ids, path_types, node_features):
    B = ids.shape[0]
    bpw = B // NW
    assert B % NW == 0 and bpw <= CH and bpw % 8 == 0

    @functools.partial(
        pl.kernel,
        out_type=(
            jax.ShapeDtypeStruct((B, K), jnp.int32),
            jax.ShapeDtypeStruct((B, K), jnp.int32),
            jax.ShapeDtypeStruct((B, D), jnp.float32),
        ),
        mesh=_sc_mesh(),
        scratch_types=[
            pltpu.VMEM((bpw,), jnp.int32),
            pltpu.VMEM((bpw, K), jnp.int32),
            pltpu.VMEM((bpw, K), jnp.int32),
            pltpu.VMEM((bpw, D), jnp.float32),
            pltpu.SemaphoreType.DMA,
            pltpu.SemaphoreType.DMA,
            pltpu.SemaphoreType.DMA,
        ],
    )
    def k(ids_h, nb_h, pt_h, ft_h, n1_o, pt0_o, x0_o,
          idx_v, n1_v, pt_v, x_v, s1, s2, s3):
        base = _wid() * bpw
        pltpu.sync_copy(ids_h.at[pl.ds(base, bpw)], idx_v)
        c1 = pltpu.async_copy(nb_h.at[idx_v], n1_v, s1)
        c2 = pltpu.async_copy(pt_h.at[idx_v], pt_v, s2)
        c3 = pltpu.async_copy(ft_h.at[idx_v], x_v, s3)
        c1.wait()
        c2.wait()
        c3.wait()
        pltpu.sync_copy(n1_v, n1_o.at[pl.ds(base, bpw)])
        pltpu.sync_copy(pt_v, pt0_o.at[pl.ds(base, bpw)])
        pltpu.sync_copy(x_v, x0_o.at[pl.ds(base, bpw)])

    return k(ids, neigh_ids, path_types, node_features)


def _gather_level2(idx2, neigh_ids, path_types):
    B = idx2.shape[0] * CH
    bpw = B // NW
    nch = bpw // CH
    assert bpw % CH == 0

    @functools.partial(
        pl.kernel,
        out_type=(
            jax.ShapeDtypeStruct((B, K), jnp.int32),
            jax.ShapeDtypeStruct((B, K), jnp.int32),
        ),
        mesh=_sc_mesh(),
        scratch_types=[
            pltpu.VMEM((nch, CH), jnp.int32),
            pltpu.VMEM((bpw, K), jnp.int32),
            pltpu.VMEM((bpw, K), jnp.int32),
            pltpu.SemaphoreType.DMA,
            pltpu.SemaphoreType.DMA,
        ],
    )
    def k(idx_h, nb_h, pt_h, n2_o, pt1_o, idx_v, n2_v, pt_v, s1, s2):
        base = _wid() * bpw
        pltpu.sync_copy(idx_h.at[pl.ds(base // CH, nch)], idx_v)
        for c in range(nch):
            lo = c * CH
            iv = idx_v.at[c]
            c1 = pltpu.async_copy(nb_h.at[iv], n2_v.at[pl.ds(lo, CH)], s1)
            c2 = pltpu.async_copy(pt_h.at[iv], pt_v.at[pl.ds(lo, CH)], s2)
            c1.wait()
            c2.wait()
        pltpu.sync_copy(n2_v, n2_o.at[pl.ds(base, bpw)])
        pltpu.sync_copy(pt_v, pt1_o.at[pl.ds(base, bpw)])

    return k(idx2, neigh_ids, path_types)


def _gather_features(idx, node_features, name):
    B = idx.shape[0]
    bpw = B // NW
    nch = max(1, bpw // CH)
    chunk = min(bpw, CH)
    assert bpw % 8 == 0 and (bpw <= CH or bpw % CH == 0)
    assert nch == 1 or nch % 2 == 0

    @functools.partial(
        pl.kernel,
        name=name,
        out_type=jax.ShapeDtypeStruct((B, D), jnp.float32),
        mesh=_sc_mesh(),
        scratch_types=[
            pltpu.VMEM((bpw,), jnp.int32),
            pltpu.VMEM((chunk, D), jnp.float32),
            pltpu.VMEM((chunk, D), jnp.float32),
            pltpu.SemaphoreType.DMA,
            pltpu.SemaphoreType.DMA,
        ],
    )
    def k(idx_h, ft_h, x_o, idx_v, buf0, buf1, s0, s1):
        base = _wid() * bpw
        pltpu.sync_copy(idx_h.at[pl.ds(base, bpw)], idx_v)
        pltpu.async_copy(ft_h.at[idx_v.at[pl.ds(0, chunk)]], buf0, s0)
        if nch == 1:
            pltpu.make_async_copy(ft_h.at[pl.ds(0, chunk)], buf0, s0).wait()
            pltpu.sync_copy(buf0, x_o.at[pl.ds(base, chunk)])
            return

        def body(p, _):
            c0 = 2 * p
            pltpu.async_copy(
                ft_h.at[idx_v.at[pl.ds((c0 + 1) * CH, CH)]], buf1, s1)
            pltpu.make_async_copy(ft_h.at[pl.ds(0, CH)], buf0, s0).wait()
            pltpu.sync_copy(buf0, x_o.at[pl.ds(base + c0 * CH, CH)])

            @pl.when(c0 + 2 < nch)
            def _():
                pltpu.async_copy(
                    ft_h.at[idx_v.at[pl.ds((c0 + 2) * CH, CH)]], buf0, s0)

            pltpu.make_async_copy(ft_h.at[pl.ds(0, CH)], buf1, s1).wait()
            pltpu.sync_copy(buf1, x_o.at[pl.ds(base + (c0 + 1) * CH, CH)])
            return 0

        lax.fori_loop(0, nch // 2, body, 0)

    return k(idx, node_features)


def _amp_table(wep_ref, wp_ref, bp_ref):
    pre = jnp.dot(wep_ref[:], wp_ref[:], preferred_element_type=jnp.float32)
    return jax.nn.sigmoid(pre + bp_ref[:])


def _weighted_mean(pt, amp_table, nv_ref, rb):
    acc = None
    for k in range(K):
        oh = (pt[:, k][:, None]
              == lax.broadcasted_iota(jnp.int32, (rb, D), 1)).astype(jnp.float32)
        ampk = jnp.dot(oh, amp_table, preferred_element_type=jnp.float32)
        term = ampk * nv_ref[:, k, :]
        acc = term if acc is None else acc + term
    return acc * (1.0 / K)


def _hop1_layer1(x2r, pt1, x1, wep, wp1, bp1, wn1, ws1, ba1):
    B2 = x1.shape[0]
    RB = 512
    grid = B2 // RB

    def body(x2_ref, pt_ref, x1_ref, wep_ref, wp1_ref, bp1_ref,
             wn1_ref, ws1_ref, ba1_ref, h1_ref):
        amp = _amp_table(wep_ref, wp1_ref, bp1_ref)
        nm = _weighted_mean(pt_ref[:], amp, x2_ref, RB)
        h = (jnp.dot(nm, wn1_ref[:], preferred_element_type=jnp.float32)
             + jnp.dot(x1_ref[:], ws1_ref[:], preferred_element_type=jnp.float32)
             + ba1_ref[:])
        h1_ref[:] = jnp.maximum(h, 0.0)

    return pl.pallas_call(
        body,
        grid=(grid,),
        in_specs=[
            pl.BlockSpec((RB, K, D), lambda i: (i, 0, 0)),
            pl.BlockSpec((RB, K), lambda i: (i, 0)),
            pl.BlockSpec((RB, D), lambda i: (i, 0)),
            pl.BlockSpec((D, WD), lambda i: (0, 0)),
            pl.BlockSpec((WD, D), lambda i: (0, 0)),
            pl.BlockSpec((1, D), lambda i: (0, 0)),
            pl.BlockSpec((D, D), lambda i: (0, 0)),
            pl.BlockSpec((D, D), lambda i: (0, 0)),
            pl.BlockSpec((1, D), lambda i: (0, 0)),
        ],
        out_specs=pl.BlockSpec((RB, D), lambda i: (i, 0)),
        out_shape=jax.ShapeDtypeStruct((B2, D), jnp.float32),
    )(x2r, pt1, x1, wep, wp1, bp1, wn1, ws1, ba1)


def _final_layer(x0, pt0, x1r, h1r, wep, wp1, bp1, wn1, ws1, ba1,
                 wp2, bp2, wn2, ws2, ba2):
    B = x0.shape[0]
    RB = 512
    grid = B // RB

    def body(x0_ref, pt_ref, x1_ref, h1_ref, wep_ref, wp1_ref, bp1_ref,
             wn1_ref, ws1_ref, ba1_ref, wp2_ref, bp2_ref, wn2_ref,
             ws2_ref, ba2_ref, out_ref):
        pt = pt_ref[:]
        amp1 = _amp_table(wep_ref, wp1_ref, bp1_ref)
        nm0 = _weighted_mean(pt, amp1, x1_ref, RB)
        h0 = (jnp.dot(nm0, wn1_ref[:], preferred_element_type=jnp.float32)
              + jnp.dot(x0_ref[:], ws1_ref[:], preferred_element_type=jnp.float32)
              + ba1_ref[:])
        h0 = jnp.maximum(h0, 0.0)
        amp2 = _amp_table(wep_ref, wp2_ref, bp2_ref)
        nm1 = _weighted_mean(pt, amp2, h1_ref, RB)
        out = (jnp.dot(nm1, wn2_ref[:], preferred_element_type=jnp.float32)
               + jnp.dot(h0, ws2_ref[:], preferred_element_type=jnp.float32)
               + ba2_ref[:])
        nrm = jnp.sqrt(jnp.sum(out * out, axis=1, keepdims=True))
        out_ref[:] = out / jnp.maximum(nrm, 1e-12)

    return pl.pallas_call(
        body,
        grid=(grid,),
        in_specs=[
            pl.BlockSpec((RB, D), lambda i: (i, 0)),
            pl.BlockSpec((RB, K), lambda i: (i, 0)),
            pl.BlockSpec((RB, K, D), lambda i: (i, 0, 0)),
            pl.BlockSpec((RB, K, D), lambda i: (i, 0, 0)),
            pl.BlockSpec((D, WD), lambda i: (0, 0)),
            pl.BlockSpec((WD, D), lambda i: (0, 0)),
            pl.BlockSpec((1, D), lambda i: (0, 0)),
            pl.BlockSpec((D, D), lambda i: (0, 0)),
            pl.BlockSpec((D, D), lambda i: (0, 0)),
            pl.BlockSpec((1, D), lambda i: (0, 0)),
            pl.BlockSpec((WD, D), lambda i: (0, 0)),
            pl.BlockSpec((1, D), lambda i: (0, 0)),
            pl.BlockSpec((D, D), lambda i: (0, 0)),
            pl.BlockSpec((D, D), lambda i: (0, 0)),
            pl.BlockSpec((1, D), lambda i: (0, 0)),
        ],
        out_specs=pl.BlockSpec((RB, D), lambda i: (i, 0)),
        out_shape=jax.ShapeDtypeStruct((B, D), jnp.float32),
    )(x0, pt0, x1r, h1r, wep, wp1, bp1, wn1, ws1, ba1,
      wp2, bp2, wn2, ws2, ba2)


def kernel(batch_keys, batch_labels, batch_negs, path_types, neigh_ids,
           node_features, walk_embeddings, weight_self_1, weight_neigh_1,
           weight_path_1, bias_path_1, bias_aggregate_1, weight_self_2,
           weight_neigh_2, weight_path_2, bias_path_2, bias_aggregate_2):
    B0 = batch_keys.shape[0]
    ids = jnp.concatenate([batch_keys, batch_labels, batch_negs])
    B = ids.shape[0]

    n1 = jnp.take(neigh_ids, ids, axis=0)
    pt0 = jnp.take(path_types, ids, axis=0)
    x0 = _gather_features(ids, node_features, "sc_gather_x0")
    n1f = n1.reshape(B * K)
    n2 = jnp.take(neigh_ids, n1f, axis=0)
    pt1 = jnp.take(path_types, n1f, axis=0)
    x1 = _gather_features(n1f, node_features, "sc_gather_x1")
    x2 = _gather_features(n2.reshape(B * K * K), node_features, "sc_gather_x2")

    wep = jnp.zeros((D, WD), jnp.float32).at[:walk_embeddings.shape[0]].set(
        walk_embeddings)
    bp1 = bias_path_1.reshape(1, D)
    ba1 = bias_aggregate_1.reshape(1, D)
    bp2 = bias_path_2.reshape(1, D)
    ba2 = bias_aggregate_2.reshape(1, D)

    h1 = _hop1_layer1(x2.reshape(B * K, K, D), pt1, x1, wep,
                      weight_path_1, bp1, weight_neigh_1, weight_self_1, ba1)
    out = _final_layer(x0, pt0, x1.reshape(B, K, D), h1.reshape(B, K, D),
                       wep, weight_path_1, bp1, weight_neigh_1,
                       weight_self_1, ba1, weight_path_2, bp2,
                       weight_neigh_2, weight_self_2, ba2)
    return (out[:B0], out[B0:2 * B0], out[2 * B0:])


# speedup vs baseline: 5.4686x; 1.0210x over previous
import functools

import jax
import jax.numpy as jnp
from jax import lax
from jax.experimental import pallas as pl
from jax.experimental.pallas import tpu as pltpu
from jax.experimental.pallas import tpu_sc as plsc

NC = 2
NS = 16
NW = NC * NS
CH = 128

K = 16
D = 128
WD = 32


def _wid():
    return lax.axis_index("s") * NC + lax.axis_index("c")


def _sc_mesh():
    return plsc.VectorSubcoreMesh(core_axis_name="c", subcore_axis_name="s")


def _gather_level1(ids, neigh_ids, path_types, node_features):
    B = ids.shape[0]
    bpw = B // NW
    assert B % NW == 0 and bpw <= CH and bpw % 8 == 0

    @functools.partial(
        pl.kernel,
        out_type=(
            jax.ShapeDtypeStruct((B, K), jnp.int32),
            jax.ShapeDtypeStruct((B, K), jnp.int32),
            jax.ShapeDtypeStruct((B, D), jnp.float32),
        ),
        mesh=_sc_mesh(),
        scratch_types=[
            pltpu.VMEM((bpw,), jnp.int32),
            pltpu.VMEM((bpw, K), jnp.int32),
            pltpu.VMEM((bpw, K), jnp.int32),
            pltpu.VMEM((bpw, D), jnp.float32),
            pltpu.SemaphoreType.DMA,
            pltpu.SemaphoreType.DMA,
            pltpu.SemaphoreType.DMA,
        ],
    )
    def k(ids_h, nb_h, pt_h, ft_h, n1_o, pt0_o, x0_o,
          idx_v, n1_v, pt_v, x_v, s1, s2, s3):
        base = _wid() * bpw
        pltpu.sync_copy(ids_h.at[pl.ds(base, bpw)], idx_v)
        c1 = pltpu.async_copy(nb_h.at[idx_v], n1_v, s1)
        c2 = pltpu.async_copy(pt_h.at[idx_v], pt_v, s2)
        c3 = pltpu.async_copy(ft_h.at[idx_v], x_v, s3)
        c1.wait()
        c2.wait()
        c3.wait()
        pltpu.sync_copy(n1_v, n1_o.at[pl.ds(base, bpw)])
        pltpu.sync_copy(pt_v, pt0_o.at[pl.ds(base, bpw)])
        pltpu.sync_copy(x_v, x0_o.at[pl.ds(base, bpw)])

    return k(ids, neigh_ids, path_types, node_features)


def _gather_level2(idx2, neigh_ids, path_types):
    B = idx2.shape[0] * CH
    bpw = B // NW
    nch = bpw // CH
    assert bpw % CH == 0

    @functools.partial(
        pl.kernel,
        out_type=(
            jax.ShapeDtypeStruct((B, K), jnp.int32),
            jax.ShapeDtypeStruct((B, K), jnp.int32),
        ),
        mesh=_sc_mesh(),
        scratch_types=[
            pltpu.VMEM((nch, CH), jnp.int32),
            pltpu.VMEM((bpw, K), jnp.int32),
            pltpu.VMEM((bpw, K), jnp.int32),
            pltpu.SemaphoreType.DMA,
            pltpu.SemaphoreType.DMA,
        ],
    )
    def k(idx_h, nb_h, pt_h, n2_o, pt1_o, idx_v, n2_v, pt_v, s1, s2):
        base = _wid() * bpw
        pltpu.sync_copy(idx_h.at[pl.ds(base // CH, nch)], idx_v)
        for c in range(nch):
            lo = c * CH
            iv = idx_v.at[c]
            c1 = pltpu.async_copy(nb_h.at[iv], n2_v.at[pl.ds(lo, CH)], s1)
            c2 = pltpu.async_copy(pt_h.at[iv], pt_v.at[pl.ds(lo, CH)], s2)
            c1.wait()
            c2.wait()
        pltpu.sync_copy(n2_v, n2_o.at[pl.ds(base, bpw)])
        pltpu.sync_copy(pt_v, pt1_o.at[pl.ds(base, bpw)])

    return k(idx2, neigh_ids, path_types)


def _gather_features(idx, node_features, name):
    B = idx.shape[0]
    bpw = B // NW
    nch = max(1, bpw // CH)
    chunk = min(bpw, CH)
    assert bpw % 8 == 0 and (bpw <= CH or bpw % CH == 0)
    assert nch == 1 or nch % 2 == 0

    @functools.partial(
        pl.kernel,
        name=name,
        out_type=jax.ShapeDtypeStruct((B, D), jnp.float32),
        mesh=_sc_mesh(),
        scratch_types=[
            pltpu.VMEM((bpw,), jnp.int32),
            pltpu.VMEM((chunk, D), jnp.float32),
            pltpu.VMEM((chunk, D), jnp.float32),
            pltpu.SemaphoreType.DMA,
            pltpu.SemaphoreType.DMA,
        ],
    )
    def k(idx_h, ft_h, x_o, idx_v, buf0, buf1, s0, s1):
        base = _wid() * bpw
        pltpu.sync_copy(idx_h.at[pl.ds(base, bpw)], idx_v)
        pltpu.async_copy(ft_h.at[idx_v.at[pl.ds(0, chunk)]], buf0, s0)
        if nch == 1:
            pltpu.make_async_copy(ft_h.at[pl.ds(0, chunk)], buf0, s0).wait()
            pltpu.sync_copy(buf0, x_o.at[pl.ds(base, chunk)])
            return

        def body(p, _):
            c0 = 2 * p
            pltpu.async_copy(
                ft_h.at[idx_v.at[pl.ds((c0 + 1) * CH, CH)]], buf1, s1)
            pltpu.make_async_copy(ft_h.at[pl.ds(0, CH)], buf0, s0).wait()
            pltpu.sync_copy(buf0, x_o.at[pl.ds(base + c0 * CH, CH)])

            @pl.when(c0 + 2 < nch)
            def _():
                pltpu.async_copy(
                    ft_h.at[idx_v.at[pl.ds((c0 + 2) * CH, CH)]], buf0, s0)

            pltpu.make_async_copy(ft_h.at[pl.ds(0, CH)], buf1, s1).wait()
            pltpu.sync_copy(buf1, x_o.at[pl.ds(base + (c0 + 1) * CH, CH)])
            return 0

        lax.fori_loop(0, nch // 2, body, 0)

    return k(idx, node_features)


def _amp_table(wep_ref, wp_ref, bp_ref):
    pre = jnp.dot(wep_ref[:], wp_ref[:], preferred_element_type=jnp.float32)
    return jax.nn.sigmoid(pre + bp_ref[:])


def _weighted_mean(pt, amp_table, nv_ref, rb):
    acc = None
    for k in range(K):
        oh = (pt[:, k][:, None]
              == lax.broadcasted_iota(jnp.int32, (rb, D), 1)).astype(jnp.float32)
        ampk = jnp.dot(oh, amp_table, preferred_element_type=jnp.float32)
        term = ampk * nv_ref[:, k, :]
        acc = term if acc is None else acc + term
    return acc * (1.0 / K)


def _hop1_layer1(x2r, pt1, x1, wep, wp1, bp1, wn1, ws1, ba1):
    B2 = x1.shape[0]
    RB = 512
    grid = B2 // RB

    def body(x2_ref, pt_ref, x1_ref, wep_ref, wp1_ref, bp1_ref,
             wn1_ref, ws1_ref, ba1_ref, h1_ref):
        amp = _amp_table(wep_ref, wp1_ref, bp1_ref)
        nm = _weighted_mean(pt_ref[:], amp, x2_ref, RB)
        h = (jnp.dot(nm, wn1_ref[:], preferred_element_type=jnp.float32)
             + jnp.dot(x1_ref[:], ws1_ref[:], preferred_element_type=jnp.float32)
             + ba1_ref[:])
        h1_ref[:] = jnp.maximum(h, 0.0)

    return pl.pallas_call(
        body,
        grid=(grid,),
        in_specs=[
            pl.BlockSpec((RB, K, D), lambda i: (i, 0, 0)),
            pl.BlockSpec((RB, K), lambda i: (i, 0)),
            pl.BlockSpec((RB, D), lambda i: (i, 0)),
            pl.BlockSpec((D, WD), lambda i: (0, 0)),
            pl.BlockSpec((WD, D), lambda i: (0, 0)),
            pl.BlockSpec((1, D), lambda i: (0, 0)),
            pl.BlockSpec((D, D), lambda i: (0, 0)),
            pl.BlockSpec((D, D), lambda i: (0, 0)),
            pl.BlockSpec((1, D), lambda i: (0, 0)),
        ],
        out_specs=pl.BlockSpec((RB, D), lambda i: (i, 0)),
        out_shape=jax.ShapeDtypeStruct((B2, D), jnp.float32),
    )(x2r, pt1, x1, wep, wp1, bp1, wn1, ws1, ba1)


def _final_layer(x0, pt0, x1r, h1r, wep, wp1, bp1, wn1, ws1, ba1,
                 wp2, bp2, wn2, ws2, ba2):
    B = x0.shape[0]
    RB = 512
    grid = B // RB

    def body(x0_ref, pt_ref, x1_ref, h1_ref, wep_ref, wp1_ref, bp1_ref,
             wn1_ref, ws1_ref, ba1_ref, wp2_ref, bp2_ref, wn2_ref,
             ws2_ref, ba2_ref, out_ref):
        pt = pt_ref[:]
        amp1 = _amp_table(wep_ref, wp1_ref, bp1_ref)
        nm0 = _weighted_mean(pt, amp1, x1_ref, RB)
        h0 = (jnp.dot(nm0, wn1_ref[:], preferred_element_type=jnp.float32)
              + jnp.dot(x0_ref[:], ws1_ref[:], preferred_element_type=jnp.float32)
              + ba1_ref[:])
        h0 = jnp.maximum(h0, 0.0)
        amp2 = _amp_table(wep_ref, wp2_ref, bp2_ref)
        nm1 = _weighted_mean(pt, amp2, h1_ref, RB)
        out = (jnp.dot(nm1, wn2_ref[:], preferred_element_type=jnp.float32)
               + jnp.dot(h0, ws2_ref[:], preferred_element_type=jnp.float32)
               + ba2_ref[:])
        nrm = jnp.sqrt(jnp.sum(out * out, axis=1, keepdims=True))
        out_ref[:] = out / jnp.maximum(nrm, 1e-12)

    return pl.pallas_call(
        body,
        grid=(grid,),
        in_specs=[
            pl.BlockSpec((RB, D), lambda i: (i, 0)),
            pl.BlockSpec((RB, K), lambda i: (i, 0)),
            pl.BlockSpec((RB, K, D), lambda i: (i, 0, 0)),
            pl.BlockSpec((RB, K, D), lambda i: (i, 0, 0)),
            pl.BlockSpec((D, WD), lambda i: (0, 0)),
            pl.BlockSpec((WD, D), lambda i: (0, 0)),
            pl.BlockSpec((1, D), lambda i: (0, 0)),
            pl.BlockSpec((D, D), lambda i: (0, 0)),
            pl.BlockSpec((D, D), lambda i: (0, 0)),
            pl.BlockSpec((1, D), lambda i: (0, 0)),
            pl.BlockSpec((WD, D), lambda i: (0, 0)),
            pl.BlockSpec((1, D), lambda i: (0, 0)),
            pl.BlockSpec((D, D), lambda i: (0, 0)),
            pl.BlockSpec((D, D), lambda i: (0, 0)),
            pl.BlockSpec((1, D), lambda i: (0, 0)),
        ],
        out_specs=pl.BlockSpec((RB, D), lambda i: (i, 0)),
        out_shape=jax.ShapeDtypeStruct((B, D), jnp.float32),
    )(x0, pt0, x1r, h1r, wep, wp1, bp1, wn1, ws1, ba1,
      wp2, bp2, wn2, ws2, ba2)


def kernel(batch_keys, batch_labels, batch_negs, path_types, neigh_ids,
           node_features, walk_embeddings, weight_self_1, weight_neigh_1,
           weight_path_1, bias_path_1, bias_aggregate_1, weight_self_2,
           weight_neigh_2, weight_path_2, bias_path_2, bias_aggregate_2):
    B0 = batch_keys.shape[0]
    ids = jnp.concatenate([batch_keys, batch_labels, batch_negs])
    B = ids.shape[0]

    n1 = jnp.take(neigh_ids, ids, axis=0)
    pt0 = jnp.take(path_types, ids, axis=0)
    x0 = _gather_features(ids, node_features, "sc_gather_x0")
    n1f = n1.reshape(B * K)
    n2 = jnp.take(neigh_ids, n1f, axis=0)
    pt1 = jnp.take(path_types, n1f, axis=0)
    x1 = _gather_features(n1f, node_features, "sc_gather_x1")
    n2f = n2.reshape(B * K * K)

    wep = jnp.zeros((D, WD), jnp.float32).at[:walk_embeddings.shape[0]].set(
        walk_embeddings)
    bp1 = bias_path_1.reshape(1, D)
    ba1 = bias_aggregate_1.reshape(1, D)
    bp2 = bias_path_2.reshape(1, D)
    ba2 = bias_aggregate_2.reshape(1, D)

    NSL = 4
    SB = B * K // NSL
    h1s = []
    for s in range(NSL):
        x2s = _gather_features(
            lax.slice_in_dim(n2f, s * SB * K, (s + 1) * SB * K),
            node_features, f"sc_gather_x2_{s}")
        h1s.append(_hop1_layer1(
            x2s.reshape(SB, K, D),
            lax.slice_in_dim(pt1, s * SB, (s + 1) * SB),
            lax.slice_in_dim(x1, s * SB, (s + 1) * SB),
            wep, weight_path_1, bp1, weight_neigh_1, weight_self_1, ba1))
    h1 = jnp.concatenate(h1s, axis=0)
    out = _final_layer(x0, pt0, x1.reshape(B, K, D), h1.reshape(B, K, D),
                       wep, weight_path_1, bp1, weight_neigh_1,
                       weight_self_1, ba1, weight_path_2, bp2,
                       weight_neigh_2, weight_self_2, ba2)
    return (out[:B0], out[B0:2 * B0], out[2 * B0:])
